# baseline (speedup 1.0000x reference)
import functools

import jax
import jax.numpy as jnp
from jax import lax
from jax.experimental import pallas as pl
from jax.experimental.pallas import tpu as pltpu
from jax.experimental.pallas import tpu_sc as plsc

N = 10000
E = 320000
H = 128

NPAD = 10240
NCORES = 2
NSUB = 16
NTILES = NCORES * NSUB
CHUNK = 128
EPT = 10240
NCHUNK = EPT // CHUNK
EPAD = NTILES * EPT
RPT = NPAD // NSUB
RB = 1024
PK = 16384

_mesh = plsc.VectorSubcoreMesh(core_axis_name="c", subcore_axis_name="s")


@functools.partial(
    pl.kernel,
    out_type=jax.ShapeDtypeStruct((NCORES, NPAD, 16), jnp.float32),
    mesh=_mesh,
    scratch_types=[
        pltpu.VMEM((NCHUNK, CHUNK), jnp.int32),
        pltpu.VMEM((CHUNK, 16), jnp.float32),
        pltpu.VMEM((CHUNK, 16), jnp.float32),
        pltpu.VMEM_SHARED((NPAD, 16), jnp.float32),
    ],
)
def _deg_kernel(dsti_hbm, out_hbm, dst_v, zbuf, obuf, acc):
    c = lax.axis_index("c")
    s = lax.axis_index("s")
    t = c * NSUB + s
    pltpu.sync_copy(dsti_hbm.at[t], dst_v)

    def _fill(i, _):
        zbuf[i] = jnp.zeros((16,), jnp.float32)
        obuf[i] = jnp.ones((16,), jnp.float32)
        return 0

    lax.fori_loop(0, CHUNK, _fill, 0)

    base = s * RPT
    for k in range(RPT // CHUNK):
        pltpu.sync_copy(zbuf, acc.at[pl.ds(base + k * CHUNK, CHUNK), :])
    plsc.subcore_barrier()

    def _body(j, _):
        pltpu.sync_copy(obuf, acc.at[dst_v.at[j]], add=True)
        return 0

    lax.fori_loop(0, NCHUNK, _body, 0)
    plsc.subcore_barrier()

    for k in range(RPT // CHUNK):
        pltpu.sync_copy(acc.at[pl.ds(base + k * CHUNK, CHUNK), :], zbuf)
        pltpu.sync_copy(zbuf, out_hbm.at[c, pl.ds(base + k * CHUNK, CHUNK), :])


NBUF = 3
CHT = 162
NCHT = NSUB * CHT
EPAD2 = NCHT * CHUNK
NSC = 10112
RSC = NSC // NSUB


@functools.partial(
    pl.kernel,
    out_type=jax.ShapeDtypeStruct((NCORES, NSC, H), jnp.float32),
    mesh=_mesh,
    scratch_types=[
        pltpu.VMEM((NBUF, CHUNK), jnp.int32),
        pltpu.VMEM((NBUF, CHUNK), jnp.int32),
        pltpu.VMEM((CHUNK, H), jnp.float32),
        pltpu.VMEM((CHUNK, H), jnp.float32),
        pltpu.VMEM((CHUNK, H), jnp.float32),
        pltpu.VMEM_SHARED((NSC, H), jnp.float32),
        [pltpu.SemaphoreType.DMA] * 3,
        [pltpu.SemaphoreType.DMA] * 3,
    ],
)
def _agg_kernel(table_hbm, pki_hbm, out_hbm, ring, dr,
                buf0, buf1, buf2, acc, isems, gsems):
    bufs = (buf0, buf1, buf2)
    c = lax.axis_index("c")
    s = lax.axis_index("s")

    def _zero(i, _):
        for cc in range(H // 16):
            bufs[0][i, pl.ds(cc * 16, 16)] = jnp.zeros((16,), jnp.float32)
        return 0

    lax.fori_loop(0, CHUNK, _zero, 0)

    base = s * RSC
    for k, rows in enumerate((128, 128, 128, 128, 120)):
        pltpu.sync_copy(bufs[0].at[pl.ds(0, rows), :],
                        acc.at[pl.ds(base + k * CHUNK, rows), :])
    plsc.subcore_barrier()

    def _unpack(slot):
        for k in range(CHUNK // 16):
            v = ring[slot, pl.ds(k * 16, 16)]
            dr[slot, pl.ds(k * 16, 16)] = lax.bitwise_and(v, PK - 1)
            ring[slot, pl.ds(k * 16, 16)] = lax.shift_right_logical(v, 14)

    @pl.when(c == 0)
    def _():
        row0 = s * CHT

        def _fetch(chunk, slot):
            pltpu.async_copy(pki_hbm.at[row0 + chunk], ring.at[slot], isems[slot])

        def _gather(slot):
            pltpu.async_copy(table_hbm.at[ring.at[slot]], bufs[slot], gsems[slot])

        for q in range(NBUF):
            _fetch(q, q)
        for q in range(2):
            pltpu.make_async_copy(pki_hbm.at[row0], ring.at[q], isems[q]).wait()
            _unpack(q)
            _gather(q)

        def _body(j0, _):
            for u in range(NBUF):
                jj = NBUF * j0 + u
                b = u
                nx = (u + 2) % NBUF
                pltpu.make_async_copy(pki_hbm.at[row0], ring.at[nx], isems[nx]).wait()
                _unpack(nx)
                _gather(nx)
                pltpu.make_async_copy(table_hbm.at[ring.at[b]], bufs[b], gsems[b]).wait()
                pltpu.sync_copy(bufs[b], acc.at[dr.at[b]], add=True)
                _fetch(lax.rem(jj + NBUF, CHT), b)
            return 0

        lax.fori_loop(0, CHT // NBUF, _body, 0)
        pltpu.make_async_copy(pki_hbm.at[row0], ring.at[2], isems[2]).wait()
        for q in range(2):
            pltpu.make_async_copy(table_hbm.at[ring.at[q]], bufs[q], gsems[q]).wait()

    plsc.subcore_barrier()

    for k, rows in enumerate((128, 128, 128, 128, 120)):
        pltpu.sync_copy(acc.at[pl.ds(base + k * CHUNK, rows), :],
                        bufs[0].at[pl.ds(0, rows), :])
        pltpu.sync_copy(bufs[0].at[pl.ds(0, rows), :],
                        out_hbm.at[c, pl.ds(base + k * CHUNK, rows), :])


def _prep_body(cnt0, cnt1, x, w0, o_scaled, o_dinv):
    deg = cnt0[:, 0:1] + cnt1[:, 0:1] + 1.0
    dinv = lax.rsqrt(deg)
    hw = jnp.dot(x[:], w0[:], preferred_element_type=jnp.float32)
    o_scaled[:] = hw * dinv
    o_dinv[:] = jnp.broadcast_to(dinv, (RB, H))


def _gru(gx, bhh):
    r = jax.nn.sigmoid(gx[:, 0:H] + bhh[:, 0:H])
    z = jax.nn.sigmoid(gx[:, H:2 * H] + bhh[:, H:2 * H])
    n = jnp.tanh(gx[:, 2 * H:3 * H] + r * bhh[:, 2 * H:3 * H])
    return (1.0 - z) * n


def _mid_body(agg0, agg1, scaled, dinv, b, wihT, bih, bhh, w_next, o_scaled2):
    conv = dinv[:] * (agg0[:] + agg1[:] + scaled[:]) + b[:]
    a = jnp.maximum(conv, 0.0)
    gx = jnp.dot(a, wihT[:], preferred_element_type=jnp.float32) + bih[:]
    h1 = _gru(gx, bhh[:])
    hw2 = jnp.dot(h1, w_next[:], preferred_element_type=jnp.float32)
    o_scaled2[:] = hw2 * dinv[:]


def _fin_body(agg0, agg1, scaled, dinv, b, wihT, bih, bhh, linWT, linb, o):
    conv = dinv[:] * (agg0[:] + agg1[:] + scaled[:]) + b[:]
    a = jnp.maximum(conv, 0.0)
    gx = jnp.dot(a, wihT[:], preferred_element_type=jnp.float32) + bih[:]
    h2 = _gru(gx, bhh[:])
    o[:] = jnp.dot(h2, linWT[:], preferred_element_type=jnp.float32) + linb[:]


_row = pl.BlockSpec((RB, H), lambda i: (i, 0))
_row16 = pl.BlockSpec((RB, 16), lambda i: (i, 0))
_w128 = pl.BlockSpec((H, H), lambda i: (0, 0))
_w384 = pl.BlockSpec((H, 3 * H), lambda i: (0, 0))
_b128 = pl.BlockSpec((1, H), lambda i: (0, 0))
_b384 = pl.BlockSpec((1, 3 * H), lambda i: (0, 0))
_GRID = (NPAD // RB,)

_prep_call = pl.pallas_call(
    _prep_body,
    grid=_GRID,
    in_specs=[_row16, _row16, _row, _w128],
    out_specs=[_row, _row],
    out_shape=[
        jax.ShapeDtypeStruct((NPAD, H), jnp.float32),
        jax.ShapeDtypeStruct((NPAD, H), jnp.float32),
    ],
)

_mid_call = pl.pallas_call(
    _mid_body,
    grid=_GRID,
    in_specs=[_row, _row, _row, _row, _b128, _w384, _b384, _b384, _w128],
    out_specs=[_row],
    out_shape=[jax.ShapeDtypeStruct((NPAD, H), jnp.float32)],
)

_fin_call = pl.pallas_call(
    _fin_body,
    grid=_GRID,
    in_specs=[_row, _row, _row, _row, _b128, _w384, _b384, _b384, _w128, _b128],
    out_specs=[_row],
    out_shape=[jax.ShapeDtypeStruct((NPAD, H), jnp.float32)],
)


@jax.jit
def kernel(x, edge_index, conv_W0, conv_b0, conv_W1, conv_b1, gru_Wih0,
           gru_Whh0, gru_bih0, gru_bhh0, gru_Wih1, gru_Whh1, gru_bih1,
           gru_bhh1, lin_W, lin_b):
    x_pad = jnp.zeros((NPAD, H), jnp.float32).at[:N].set(x)
    pki = jnp.concatenate(
        [edge_index[0] * PK + edge_index[1],
         jnp.full((EPAD2 - E,), N, jnp.int32)]).reshape(NCHT, CHUNK)
    dst3 = jnp.concatenate(
        [edge_index[1], jnp.full((EPAD - E,), N, jnp.int32)]).reshape(
            NTILES, NCHUNK, CHUNK)
    b0 = conv_b0.reshape(1, H)
    b1 = conv_b1.reshape(1, H)
    wih0T = gru_Wih0.T
    wih1T = gru_Wih1.T
    bih0 = gru_bih0.reshape(1, 3 * H)
    bhh0 = gru_bhh0.reshape(1, 3 * H)
    bih1 = gru_bih1.reshape(1, 3 * H)
    bhh1 = gru_bhh1.reshape(1, 3 * H)
    linWT = jnp.zeros((H, H), jnp.float32).at[:, :2].set(lin_W.T)
    linb = jnp.zeros((1, H), jnp.float32).at[:, :2].set(lin_b.reshape(1, 2))

    cnt = _deg_kernel(dst3)
    scaled1, dinv = _prep_call(cnt[0], cnt[1], x_pad, conv_W0)
    agg1 = jnp.pad(_agg_kernel(scaled1, pki), ((0, 0), (0, NPAD - NSC), (0, 0)))
    (scaled2,) = _mid_call(agg1[0], agg1[1], scaled1, dinv, b0, wih0T, bih0,
                           bhh0, conv_W1)
    agg2 = jnp.pad(_agg_kernel(scaled2, pki), ((0, 0), (0, NPAD - NSC), (0, 0)))
    (res,) = _fin_call(agg2[0], agg2[1], scaled2, dinv, b1, wih1T, bih1,
                       bhh1, linWT, linb)
    return res[:N, :2]


# speedup vs baseline: 1.1632x; 1.1632x over previous
import functools

import jax
import jax.numpy as jnp
from jax import lax
from jax.experimental import pallas as pl
from jax.experimental.pallas import tpu as pltpu
from jax.experimental.pallas import tpu_sc as plsc

N = 10000
E = 320000
H = 128

NPAD = 10240
NCORES = 2
NSUB = 16
NTILES = NCORES * NSUB
CHUNK = 128
EPT = 10240
NCHUNK = EPT // CHUNK
EPAD = NTILES * EPT
RPT = NPAD // NSUB
RB = 1024
PK = 16384

_mesh = plsc.VectorSubcoreMesh(core_axis_name="c", subcore_axis_name="s")


@functools.partial(
    pl.kernel,
    out_type=jax.ShapeDtypeStruct((NCORES, NPAD, 16), jnp.float32),
    mesh=_mesh,
    scratch_types=[
        pltpu.VMEM((NCHUNK, CHUNK), jnp.int32),
        pltpu.VMEM((CHUNK, 16), jnp.float32),
        pltpu.VMEM((CHUNK, 16), jnp.float32),
        pltpu.VMEM_SHARED((NPAD, 16), jnp.float32),
    ],
)
def _deg_kernel(dsti_hbm, out_hbm, dst_v, zbuf, obuf, acc):
    c = lax.axis_index("c")
    s = lax.axis_index("s")
    t = c * NSUB + s
    pltpu.sync_copy(dsti_hbm.at[t], dst_v)

    def _fill(i, _):
        zbuf[i] = jnp.zeros((16,), jnp.float32)
        obuf[i] = jnp.ones((16,), jnp.float32)
        return 0

    lax.fori_loop(0, CHUNK, _fill, 0)

    base = s * RPT
    for k in range(RPT // CHUNK):
        pltpu.sync_copy(zbuf, acc.at[pl.ds(base + k * CHUNK, CHUNK), :])
    plsc.subcore_barrier()

    def _body(j, _):
        pltpu.sync_copy(obuf, acc.at[dst_v.at[j]], add=True)
        return 0

    lax.fori_loop(0, NCHUNK, _body, 0)
    plsc.subcore_barrier()

    for k in range(RPT // CHUNK):
        pltpu.sync_copy(acc.at[pl.ds(base + k * CHUNK, CHUNK), :], zbuf)
        pltpu.sync_copy(zbuf, out_hbm.at[c, pl.ds(base + k * CHUNK, CHUNK), :])


IBLK = 4
CHT = 160
NBLK = CHT // IBLK
NCHT = NSUB * CHT
EPAD2 = NCHT * CHUNK
NSC = 10112
RSC = NSC // NSUB
TRASH = 10016


@functools.partial(
    pl.kernel,
    out_type=jax.ShapeDtypeStruct((NCORES, NSC, H), jnp.float32),
    mesh=_mesh,
    scratch_types=[
        pltpu.VMEM((2, IBLK, CHUNK), jnp.int32),
        pltpu.VMEM((2, CHUNK), jnp.int32),
        pltpu.VMEM((2, CHUNK), jnp.int32),
        pltpu.VMEM((CHUNK, H), jnp.float32),
        pltpu.VMEM((CHUNK, H), jnp.float32),
        pltpu.VMEM_SHARED((NSC, H), jnp.float32),
        [pltpu.SemaphoreType.DMA] * 2,
        [pltpu.SemaphoreType.DMA] * 2,
    ],
)
def _agg_kernel(table_hbm, pki_hbm, out_hbm, iblk, sr, dr,
                buf0, buf1, acc, bsems, gsems):
    bufs = (buf0, buf1)
    c = lax.axis_index("c")
    s = lax.axis_index("s")

    def _zero(i, _):
        for cc in range(H // 16):
            bufs[0][i, pl.ds(cc * 16, 16)] = jnp.zeros((16,), jnp.float32)
        return 0

    lax.fori_loop(0, CHUNK, _zero, 0)

    base = s * RSC
    for k, rows in enumerate((128, 128, 128, 128, 120)):
        pltpu.sync_copy(bufs[0].at[pl.ds(0, rows), :],
                        acc.at[pl.ds(base + k * CHUNK, rows), :])
    plsc.subcore_barrier()

    @pl.when(c == 0)
    def _():
        row0 = s * CHT

        def _fetch_blk(m, slot):
            pltpu.async_copy(pki_hbm.at[pl.ds(row0 + m * IBLK, IBLK), :],
                             iblk.at[slot], bsems[slot])

        def _wait_blk(slot):
            pltpu.make_async_copy(pki_hbm.at[pl.ds(row0, IBLK), :],
                                  iblk.at[slot], bsems[slot]).wait()

        def _wait_gather(b):
            pltpu.make_async_copy(table_hbm.at[sr.at[b]], bufs[b],
                                  gsems[b]).wait()

        for b in range(2):
            for k in range(CHUNK // 16):
                sr[b, pl.ds(k * 16, 16)] = jnp.zeros((16,), jnp.int32)
                dr[b, pl.ds(k * 16, 16)] = jnp.full((16,), TRASH, jnp.int32)
            _fetch_blk(b, b)
            pltpu.async_copy(table_hbm.at[sr.at[b]], bufs[b], gsems[b])

        def _step(slot, r, b):
            _wait_gather(b)
            pltpu.sync_copy(bufs[b], acc.at[dr.at[b]], add=True)
            for k in range(CHUNK // 16):
                v = iblk[slot, r, pl.ds(k * 16, 16)]
                sr[b, pl.ds(k * 16, 16)] = lax.shift_right_logical(v, 14)
                dr[b, pl.ds(k * 16, 16)] = lax.bitwise_and(v, PK - 1)
            pltpu.async_copy(table_hbm.at[sr.at[b]], bufs[b], gsems[b])

        def _body(bp, _):
            for slot in range(2):
                _wait_blk(slot)
                for r in range(IBLK):
                    _step(slot, r, r % 2)
                _fetch_blk(lax.rem(2 * bp + slot + 2, NBLK), slot)
            return 0

        lax.fori_loop(0, NBLK // 2, _body, 0)
        for b in range(2):
            _wait_gather(b)
            pltpu.sync_copy(bufs[b], acc.at[dr.at[b]], add=True)
            _wait_blk(b)

    plsc.subcore_barrier()

    for k, rows in enumerate((128, 128, 128, 128, 120)):
        pltpu.sync_copy(acc.at[pl.ds(base + k * CHUNK, rows), :],
                        bufs[0].at[pl.ds(0, rows), :])
        pltpu.sync_copy(bufs[0].at[pl.ds(0, rows), :],
                        out_hbm.at[c, pl.ds(base + k * CHUNK, rows), :])


def _prep_body(cnt0, cnt1, x, w0, o_scaled, o_dinv):
    deg = cnt0[:, 0:1] + cnt1[:, 0:1] + 1.0
    dinv = lax.rsqrt(deg)
    hw = jnp.dot(x[:], w0[:], preferred_element_type=jnp.float32)
    o_scaled[:] = hw * dinv
    o_dinv[:] = jnp.broadcast_to(dinv, (RB, H))


def _gru(gx, bhh):
    r = jax.nn.sigmoid(gx[:, 0:H] + bhh[:, 0:H])
    z = jax.nn.sigmoid(gx[:, H:2 * H] + bhh[:, H:2 * H])
    n = jnp.tanh(gx[:, 2 * H:3 * H] + r * bhh[:, 2 * H:3 * H])
    return (1.0 - z) * n


def _mid_body(agg0, agg1, scaled, dinv, b, wihT, bih, bhh, w_next, o_scaled2):
    conv = dinv[:] * (agg0[:] + agg1[:] + scaled[:]) + b[:]
    a = jnp.maximum(conv, 0.0)
    gx = jnp.dot(a, wihT[:], preferred_element_type=jnp.float32) + bih[:]
    h1 = _gru(gx, bhh[:])
    hw2 = jnp.dot(h1, w_next[:], preferred_element_type=jnp.float32)
    o_scaled2[:] = hw2 * dinv[:]


def _fin_body(agg0, agg1, scaled, dinv, b, wihT, bih, bhh, linWT, linb, o):
    conv = dinv[:] * (agg0[:] + agg1[:] + scaled[:]) + b[:]
    a = jnp.maximum(conv, 0.0)
    gx = jnp.dot(a, wihT[:], preferred_element_type=jnp.float32) + bih[:]
    h2 = _gru(gx, bhh[:])
    o[:] = jnp.dot(h2, linWT[:], preferred_element_type=jnp.float32) + linb[:]


_row = pl.BlockSpec((RB, H), lambda i: (i, 0))
_row16 = pl.BlockSpec((RB, 16), lambda i: (i, 0))
_w128 = pl.BlockSpec((H, H), lambda i: (0, 0))
_w384 = pl.BlockSpec((H, 3 * H), lambda i: (0, 0))
_b128 = pl.BlockSpec((1, H), lambda i: (0, 0))
_b384 = pl.BlockSpec((1, 3 * H), lambda i: (0, 0))
_GRID = (NPAD // RB,)

_prep_call = pl.pallas_call(
    _prep_body,
    grid=_GRID,
    in_specs=[_row16, _row16, _row, _w128],
    out_specs=[_row, _row],
    out_shape=[
        jax.ShapeDtypeStruct((NPAD, H), jnp.float32),
        jax.ShapeDtypeStruct((NPAD, H), jnp.float32),
    ],
)

_mid_call = pl.pallas_call(
    _mid_body,
    grid=_GRID,
    in_specs=[_row, _row, _row, _row, _b128, _w384, _b384, _b384, _w128],
    out_specs=[_row],
    out_shape=[jax.ShapeDtypeStruct((NPAD, H), jnp.float32)],
)

_fin_call = pl.pallas_call(
    _fin_body,
    grid=_GRID,
    in_specs=[_row, _row, _row, _row, _b128, _w384, _b384, _b384, _w128, _b128],
    out_specs=[_row],
    out_shape=[jax.ShapeDtypeStruct((NPAD, H), jnp.float32)],
)


@jax.jit
def kernel(x, edge_index, conv_W0, conv_b0, conv_W1, conv_b1, gru_Wih0,
           gru_Whh0, gru_bih0, gru_bhh0, gru_Wih1, gru_Whh1, gru_bih1,
           gru_bhh1, lin_W, lin_b):
    x_pad = jnp.zeros((NPAD, H), jnp.float32).at[:N].set(x)
    pki = jnp.concatenate(
        [edge_index[0] * PK + edge_index[1],
         jnp.full((EPAD2 - E,), N, jnp.int32)]).reshape(NCHT, CHUNK)
    dst3 = jnp.concatenate(
        [edge_index[1], jnp.full((EPAD - E,), N, jnp.int32)]).reshape(
            NTILES, NCHUNK, CHUNK)
    b0 = conv_b0.reshape(1, H)
    b1 = conv_b1.reshape(1, H)
    wih0T = gru_Wih0.T
    wih1T = gru_Wih1.T
    bih0 = gru_bih0.reshape(1, 3 * H)
    bhh0 = gru_bhh0.reshape(1, 3 * H)
    bih1 = gru_bih1.reshape(1, 3 * H)
    bhh1 = gru_bhh1.reshape(1, 3 * H)
    linWT = jnp.zeros((H, H), jnp.float32).at[:, :2].set(lin_W.T)
    linb = jnp.zeros((1, H), jnp.float32).at[:, :2].set(lin_b.reshape(1, 2))

    cnt = _deg_kernel(dst3)
    scaled1, dinv = _prep_call(cnt[0], cnt[1], x_pad, conv_W0)
    agg1 = jnp.pad(_agg_kernel(scaled1, pki), ((0, 0), (0, NPAD - NSC), (0, 0)))
    (scaled2,) = _mid_call(agg1[0], agg1[1], scaled1, dinv, b0, wih0T, bih0,
                           bhh0, conv_W1)
    agg2 = jnp.pad(_agg_kernel(scaled2, pki), ((0, 0), (0, NPAD - NSC), (0, 0)))
    (res,) = _fin_call(agg2[0], agg2[1], scaled2, dinv, b1, wih1T, bih1,
                       bhh1, linWT, linb)
    return res[:N, :2]


# speedup vs baseline: 1.4761x; 1.2690x over previous
import functools

import jax
import jax.numpy as jnp
from jax import lax
from jax.experimental import pallas as pl
from jax.experimental.pallas import tpu as pltpu
from jax.experimental.pallas import tpu_sc as plsc

N = 10000
E = 320000
H = 128

NPAD = 10240
NCORES = 2
NSUB = 16
NTILES = NCORES * NSUB
CHUNK = 128
EPT = 10240
NCHUNK = EPT // CHUNK
EPAD = NTILES * EPT
RPT = NPAD // NSUB
RB = 1024
PK = 16384

_mesh = plsc.VectorSubcoreMesh(core_axis_name="c", subcore_axis_name="s")


@functools.partial(
    pl.kernel,
    out_type=jax.ShapeDtypeStruct((NCORES, NPAD, 16), jnp.float32),
    mesh=_mesh,
    scratch_types=[
        pltpu.VMEM((NCHUNK, CHUNK), jnp.int32),
        pltpu.VMEM((CHUNK, 16), jnp.float32),
        pltpu.VMEM((CHUNK, 16), jnp.float32),
        pltpu.VMEM_SHARED((NPAD, 16), jnp.float32),
    ],
)
def _deg_kernel(dsti_hbm, out_hbm, dst_v, zbuf, obuf, acc):
    c = lax.axis_index("c")
    s = lax.axis_index("s")
    t = c * NSUB + s
    pltpu.sync_copy(dsti_hbm.at[t], dst_v)

    def _fill(i, _):
        zbuf[i] = jnp.zeros((16,), jnp.float32)
        obuf[i] = jnp.ones((16,), jnp.float32)
        return 0

    lax.fori_loop(0, CHUNK, _fill, 0)

    base = s * RPT
    for k in range(RPT // CHUNK):
        pltpu.sync_copy(zbuf, acc.at[pl.ds(base + k * CHUNK, CHUNK), :])
    plsc.subcore_barrier()

    def _body(j, _):
        pltpu.sync_copy(obuf, acc.at[dst_v.at[j]], add=True)
        return 0

    lax.fori_loop(0, NCHUNK, _body, 0)
    plsc.subcore_barrier()

    for k in range(RPT // CHUNK):
        pltpu.sync_copy(acc.at[pl.ds(base + k * CHUNK, CHUNK), :], zbuf)
        pltpu.sync_copy(zbuf, out_hbm.at[c, pl.ds(base + k * CHUNK, CHUNK), :])


NBUF = 2
CHT = 160
HCH = CHT // 2
NCHT = NSUB * CHT
EPAD2 = NCHT * CHUNK
NSC = 10112
RSC = NSC // NSUB


@functools.partial(
    pl.kernel,
    out_type=jax.ShapeDtypeStruct((NCORES, NSC, H), jnp.float32),
    mesh=_mesh,
    scratch_types=[
        pltpu.VMEM((HCH, CHUNK), jnp.int32),
        pltpu.VMEM((NBUF, CHUNK), jnp.int32),
        pltpu.VMEM((NBUF, CHUNK), jnp.int32),
        pltpu.VMEM((CHUNK, H), jnp.float32),
        pltpu.VMEM((CHUNK, H), jnp.float32),
        pltpu.VMEM_SHARED((NSC, H), jnp.float32),
        pltpu.SemaphoreType.DMA,
        pltpu.SemaphoreType.DMA,
    ],
)
def _agg_kernel(table_hbm, pki_hbm, out_hbm, pk_v, sr, dr,
                buf0, buf1, acc, sem0, sem1):
    bufs = (buf0, buf1)
    sems = (sem0, sem1)
    c = lax.axis_index("c")
    s = lax.axis_index("s")

    def _zero(i, _):
        for cc in range(H // 16):
            bufs[0][i, pl.ds(cc * 16, 16)] = jnp.zeros((16,), jnp.float32)
        return 0

    lax.fori_loop(0, CHUNK, _zero, 0)

    base = s * RSC
    for k, rows in enumerate((128, 128, 128, 128, 120)):
        pltpu.sync_copy(bufs[0].at[pl.ds(0, rows), :],
                        acc.at[pl.ds(base + k * CHUNK, rows), :])
    plsc.subcore_barrier()

    def _unpack(jj, b):
        for k in range(CHUNK // 16):
            v = pk_v[jj, pl.ds(k * 16, 16)]
            sr[b, pl.ds(k * 16, 16)] = lax.shift_right_logical(v, 14)
            dr[b, pl.ds(k * 16, 16)] = lax.bitwise_and(v, PK - 1)

    @pl.when(c == 0)
    def _():
        row0 = s * CHT
        for half in range(2):
            pltpu.sync_copy(
                pki_hbm.at[pl.ds(row0 + half * HCH, HCH), :], pk_v)
            for b in range(NBUF):
                _unpack(b, b)
                pltpu.async_copy(table_hbm.at[sr.at[b]], bufs[b], sems[b])

            def _body(j0, _):
                for b in range(NBUF):
                    jj = NBUF * j0 + b
                    pltpu.make_async_copy(table_hbm.at[sr.at[b]], bufs[b],
                                          sems[b]).wait()
                    pltpu.sync_copy(bufs[b], acc.at[dr.at[b]], add=True)
                    _unpack(lax.rem(jj + NBUF, HCH), b)
                    pltpu.async_copy(table_hbm.at[sr.at[b]], bufs[b], sems[b])
                return 0

            lax.fori_loop(0, HCH // NBUF, _body, 0)
            for b in range(NBUF):
                pltpu.make_async_copy(table_hbm.at[sr.at[b]], bufs[b],
                                      sems[b]).wait()

    plsc.subcore_barrier()

    for k, rows in enumerate((128, 128, 128, 128, 120)):
        pltpu.sync_copy(acc.at[pl.ds(base + k * CHUNK, rows), :],
                        bufs[0].at[pl.ds(0, rows), :])
        pltpu.sync_copy(bufs[0].at[pl.ds(0, rows), :],
                        out_hbm.at[c, pl.ds(base + k * CHUNK, rows), :])


def _prep_body(cnt0, cnt1, x, w0, o_scaled, o_dinv):
    deg = cnt0[:, 0:1] + cnt1[:, 0:1] + 1.0
    dinv = lax.rsqrt(deg)
    hw = jnp.dot(x[:], w0[:], preferred_element_type=jnp.float32)
    o_scaled[:] = hw * dinv
    o_dinv[:] = jnp.broadcast_to(dinv, (RB, H))


def _gru(gx, bhh):
    r = jax.nn.sigmoid(gx[:, 0:H] + bhh[:, 0:H])
    z = jax.nn.sigmoid(gx[:, H:2 * H] + bhh[:, H:2 * H])
    n = jnp.tanh(gx[:, 2 * H:3 * H] + r * bhh[:, 2 * H:3 * H])
    return (1.0 - z) * n


def _mid_body(agg0, agg1, scaled, dinv, b, wihT, bih, bhh, w_next, o_scaled2):
    conv = dinv[:] * (agg0[:] + agg1[:] + scaled[:]) + b[:]
    a = jnp.maximum(conv, 0.0)
    gx = jnp.dot(a, wihT[:], preferred_element_type=jnp.float32) + bih[:]
    h1 = _gru(gx, bhh[:])
    hw2 = jnp.dot(h1, w_next[:], preferred_element_type=jnp.float32)
    o_scaled2[:] = hw2 * dinv[:]


def _fin_body(agg0, agg1, scaled, dinv, b, wihT, bih, bhh, linWT, linb, o):
    conv = dinv[:] * (agg0[:] + agg1[:] + scaled[:]) + b[:]
    a = jnp.maximum(conv, 0.0)
    gx = jnp.dot(a, wihT[:], preferred_element_type=jnp.float32) + bih[:]
    h2 = _gru(gx, bhh[:])
    o[:] = jnp.dot(h2, linWT[:], preferred_element_type=jnp.float32) + linb[:]


_row = pl.BlockSpec((RB, H), lambda i: (i, 0))
_row16 = pl.BlockSpec((RB, 16), lambda i: (i, 0))
_w128 = pl.BlockSpec((H, H), lambda i: (0, 0))
_w384 = pl.BlockSpec((H, 3 * H), lambda i: (0, 0))
_b128 = pl.BlockSpec((1, H), lambda i: (0, 0))
_b384 = pl.BlockSpec((1, 3 * H), lambda i: (0, 0))
_GRID = (NPAD // RB,)

_prep_call = pl.pallas_call(
    _prep_body,
    grid=_GRID,
    in_specs=[_row16, _row16, _row, _w128],
    out_specs=[_row, _row],
    out_shape=[
        jax.ShapeDtypeStruct((NPAD, H), jnp.float32),
        jax.ShapeDtypeStruct((NPAD, H), jnp.float32),
    ],
)

_mid_call = pl.pallas_call(
    _mid_body,
    grid=_GRID,
    in_specs=[_row, _row, _row, _row, _b128, _w384, _b384, _b384, _w128],
    out_specs=[_row],
    out_shape=[jax.ShapeDtypeStruct((NPAD, H), jnp.float32)],
)

_fin_call = pl.pallas_call(
    _fin_body,
    grid=_GRID,
    in_specs=[_row, _row, _row, _row, _b128, _w384, _b384, _b384, _w128, _b128],
    out_specs=[_row],
    out_shape=[jax.ShapeDtypeStruct((NPAD, H), jnp.float32)],
)


@jax.jit
def kernel(x, edge_index, conv_W0, conv_b0, conv_W1, conv_b1, gru_Wih0,
           gru_Whh0, gru_bih0, gru_bhh0, gru_Wih1, gru_Whh1, gru_bih1,
           gru_bhh1, lin_W, lin_b):
    x_pad = jnp.zeros((NPAD, H), jnp.float32).at[:N].set(x)
    pki = jnp.concatenate(
        [edge_index[0] * PK + edge_index[1],
         jnp.full((EPAD2 - E,), N, jnp.int32)]).reshape(NCHT, CHUNK)
    dst3 = jnp.concatenate(
        [edge_index[1], jnp.full((EPAD - E,), N, jnp.int32)]).reshape(
            NTILES, NCHUNK, CHUNK)
    b0 = conv_b0.reshape(1, H)
    b1 = conv_b1.reshape(1, H)
    wih0T = gru_Wih0.T
    wih1T = gru_Wih1.T
    bih0 = gru_bih0.reshape(1, 3 * H)
    bhh0 = gru_bhh0.reshape(1, 3 * H)
    bih1 = gru_bih1.reshape(1, 3 * H)
    bhh1 = gru_bhh1.reshape(1, 3 * H)
    linWT = jnp.zeros((H, H), jnp.float32).at[:, :2].set(lin_W.T)
    linb = jnp.zeros((1, H), jnp.float32).at[:, :2].set(lin_b.reshape(1, 2))

    cnt = _deg_kernel(dst3)
    scaled1, dinv = _prep_call(cnt[0], cnt[1], x_pad, conv_W0)
    agg1 = jnp.pad(_agg_kernel(scaled1, pki), ((0, 0), (0, NPAD - NSC), (0, 0)))
    (scaled2,) = _mid_call(agg1[0], agg1[1], scaled1, dinv, b0, wih0T, bih0,
                           bhh0, conv_W1)
    agg2 = jnp.pad(_agg_kernel(scaled2, pki), ((0, 0), (0, NPAD - NSC), (0, 0)))
    (res,) = _fin_call(agg2[0], agg2[1], scaled2, dinv, b1, wih1T, bih1,
                       bhh1, linWT, linb)
    return res[:N, :2]


# speedup vs baseline: 1.5626x; 1.0587x over previous
import functools

import jax
import jax.numpy as jnp
from jax import lax
from jax.experimental import pallas as pl
from jax.experimental.pallas import tpu as pltpu
from jax.experimental.pallas import tpu_sc as plsc

N = 10000
E = 320000
H = 128

NPAD = 10240
NCORES = 2
NSUB = 16
NTILES = NCORES * NSUB
CHUNK = 128
EPT = 10240
NCHUNK = EPT // CHUNK
EPAD = NTILES * EPT
RPT = NPAD // NSUB
RB = 1024
PK = 16384

_mesh = plsc.VectorSubcoreMesh(core_axis_name="c", subcore_axis_name="s")


@functools.partial(
    pl.kernel,
    out_type=jax.ShapeDtypeStruct((NCORES, NPAD, 16), jnp.float32),
    mesh=_mesh,
    scratch_types=[
        pltpu.VMEM((NCHUNK, CHUNK), jnp.int32),
        pltpu.VMEM((CHUNK, 16), jnp.float32),
        pltpu.VMEM((CHUNK, 16), jnp.float32),
        pltpu.VMEM_SHARED((NPAD, 16), jnp.float32),
    ],
)
def _deg_kernel(dsti_hbm, z16_hbm, o16_hbm, out_hbm, dst_v, zbuf, obuf, acc):
    c = lax.axis_index("c")
    s = lax.axis_index("s")
    t = c * NSUB + s
    pltpu.sync_copy(dsti_hbm.at[pl.ds(t * NCHUNK, NCHUNK), :], dst_v)
    pltpu.sync_copy(z16_hbm, zbuf)
    pltpu.sync_copy(o16_hbm, obuf)

    base = s * RPT
    for k in range(RPT // CHUNK):
        pltpu.sync_copy(zbuf, acc.at[pl.ds(base + k * CHUNK, CHUNK), :])
    plsc.subcore_barrier()

    def _body(j, _):
        pltpu.sync_copy(obuf, acc.at[dst_v.at[j]], add=True)
        return 0

    lax.fori_loop(0, NCHUNK, _body, 0)
    plsc.subcore_barrier()

    for k in range(RPT // CHUNK):
        pltpu.sync_copy(acc.at[pl.ds(base + k * CHUNK, CHUNK), :], zbuf)
        pltpu.sync_copy(zbuf, out_hbm.at[c, pl.ds(base + k * CHUNK, CHUNK), :])


NBUF = 2
CHT = 160
PART = 40
NPART = CHT // PART
NCHT = NSUB * CHT
EPAD2 = NCHT * CHUNK
NSC = 10112
RSC = NSC // NSUB


@functools.partial(
    pl.kernel,
    out_type=jax.ShapeDtypeStruct((NCORES, NSC, H), jnp.float32),
    mesh=_mesh,
    scratch_types=[
        pltpu.VMEM((PART, CHUNK), jnp.int32),
        pltpu.VMEM((PART, CHUNK), jnp.int32),
        pltpu.VMEM((CHUNK, H), jnp.float32),
        pltpu.VMEM((CHUNK, H), jnp.float32),
        pltpu.VMEM_SHARED((NSC, H), jnp.float32),
        pltpu.SemaphoreType.DMA,
        pltpu.SemaphoreType.DMA,
    ],
)
def _agg_kernel(table_hbm, srci_hbm, dsti_hbm, z_hbm, out_hbm,
                src_v, dst_v, buf0, buf1, acc, sem0, sem1):
    bufs = (buf0, buf1)
    sems = (sem0, sem1)
    c = lax.axis_index("c")
    s = lax.axis_index("s")

    base = s * RSC
    for k, rows in enumerate((128, 128, 128, 128, 120)):
        pltpu.sync_copy(z_hbm.at[pl.ds(0, rows), :],
                        acc.at[pl.ds(base + k * CHUNK, rows), :])
    plsc.subcore_barrier()

    @pl.when(c == 0)
    def _():
        row0 = s * CHT
        for part in range(NPART):
            pltpu.sync_copy(
                srci_hbm.at[pl.ds(row0 + part * PART, PART), :], src_v)
            pltpu.sync_copy(
                dsti_hbm.at[pl.ds(row0 + part * PART, PART), :], dst_v)
            for b in range(NBUF):
                pltpu.async_copy(table_hbm.at[src_v.at[b]], bufs[b], sems[b])

            def _body(j0, _):
                for b in range(NBUF):
                    jj = NBUF * j0 + b
                    pltpu.make_async_copy(table_hbm.at[src_v.at[b]], bufs[b],
                                          sems[b]).wait()
                    pltpu.sync_copy(bufs[b], acc.at[dst_v.at[jj]], add=True)
                    nxt = lax.rem(jj + NBUF, PART)
                    pltpu.async_copy(table_hbm.at[src_v.at[nxt]], bufs[b],
                                     sems[b])
                return 0

            lax.fori_loop(0, PART // NBUF, _body, 0)
            for b in range(NBUF):
                pltpu.make_async_copy(table_hbm.at[src_v.at[b]], bufs[b],
                                      sems[b]).wait()

    plsc.subcore_barrier()

    for k, rows in enumerate((128, 128, 128, 128, 120)):
        pltpu.sync_copy(acc.at[pl.ds(base + k * CHUNK, rows), :],
                        bufs[0].at[pl.ds(0, rows), :])
        pltpu.sync_copy(bufs[0].at[pl.ds(0, rows), :],
                        out_hbm.at[c, pl.ds(base + k * CHUNK, rows), :])


def _prep_body(cnt0, cnt1, x, w0, o_scaled, o_dinv):
    deg = cnt0[:, 0:1] + cnt1[:, 0:1] + 1.0
    dinv = lax.rsqrt(deg)
    hw = jnp.dot(x[:], w0[:], preferred_element_type=jnp.float32)
    o_scaled[:] = hw * dinv
    o_dinv[:] = jnp.broadcast_to(dinv, (RB, H))


def _gru(gx, bhh):
    r = jax.nn.sigmoid(gx[:, 0:H] + bhh[:, 0:H])
    z = jax.nn.sigmoid(gx[:, H:2 * H] + bhh[:, H:2 * H])
    n = jnp.tanh(gx[:, 2 * H:3 * H] + r * bhh[:, 2 * H:3 * H])
    return (1.0 - z) * n


def _mid_body(agg0, agg1, scaled, dinv, b, wihT, bih, bhh, w_next, o_scaled2):
    conv = dinv[:] * (agg0[:] + agg1[:] + scaled[:]) + b[:]
    a = jnp.maximum(conv, 0.0)
    gx = jnp.dot(a, wihT[:], preferred_element_type=jnp.float32) + bih[:]
    h1 = _gru(gx, bhh[:])
    hw2 = jnp.dot(h1, w_next[:], preferred_element_type=jnp.float32)
    o_scaled2[:] = hw2 * dinv[:]


def _fin_body(agg0, agg1, scaled, dinv, b, wihT, bih, bhh, linWT, linb, o):
    conv = dinv[:] * (agg0[:] + agg1[:] + scaled[:]) + b[:]
    a = jnp.maximum(conv, 0.0)
    gx = jnp.dot(a, wihT[:], preferred_element_type=jnp.float32) + bih[:]
    h2 = _gru(gx, bhh[:])
    o[:] = jnp.dot(h2, linWT[:], preferred_element_type=jnp.float32) + linb[:]


_row = pl.BlockSpec((RB, H), lambda i: (i, 0))
_row16 = pl.BlockSpec((RB, 16), lambda i: (i, 0))
_w128 = pl.BlockSpec((H, H), lambda i: (0, 0))
_w384 = pl.BlockSpec((H, 3 * H), lambda i: (0, 0))
_b128 = pl.BlockSpec((1, H), lambda i: (0, 0))
_b384 = pl.BlockSpec((1, 3 * H), lambda i: (0, 0))
_GRID = (NPAD // RB,)

_prep_call = pl.pallas_call(
    _prep_body,
    grid=_GRID,
    in_specs=[_row16, _row16, _row, _w128],
    out_specs=[_row, _row],
    out_shape=[
        jax.ShapeDtypeStruct((NPAD, H), jnp.float32),
        jax.ShapeDtypeStruct((NPAD, H), jnp.float32),
    ],
)

_mid_call = pl.pallas_call(
    _mid_body,
    grid=_GRID,
    in_specs=[_row, _row, _row, _row, _b128, _w384, _b384, _b384, _w128],
    out_specs=[_row],
    out_shape=[jax.ShapeDtypeStruct((NPAD, H), jnp.float32)],
)

_fin_call = pl.pallas_call(
    _fin_body,
    grid=_GRID,
    in_specs=[_row, _row, _row, _row, _b128, _w384, _b384, _b384, _w128, _b128],
    out_specs=[_row],
    out_shape=[jax.ShapeDtypeStruct((NPAD, H), jnp.float32)],
)


@jax.jit
def kernel(x, edge_index, conv_W0, conv_b0, conv_W1, conv_b1, gru_Wih0,
           gru_Whh0, gru_bih0, gru_bhh0, gru_Wih1, gru_Whh1, gru_bih1,
           gru_bhh1, lin_W, lin_b):
    x_pad = jnp.zeros((NPAD, H), jnp.float32).at[:N].set(x)
    srci = jnp.concatenate(
        [edge_index[0], jnp.zeros((EPAD2 - E,), jnp.int32)]).reshape(
            NCHT, CHUNK)
    dsti = jnp.concatenate(
        [edge_index[1], jnp.full((EPAD2 - E,), N, jnp.int32)]).reshape(
            NCHT, CHUNK)
    z128 = jnp.zeros((CHUNK, H), jnp.float32)
    z16 = jnp.zeros((CHUNK, 16), jnp.float32)
    o16 = jnp.ones((CHUNK, 16), jnp.float32)
    b0 = conv_b0.reshape(1, H)
    b1 = conv_b1.reshape(1, H)
    wih0T = gru_Wih0.T
    wih1T = gru_Wih1.T
    bih0 = gru_bih0.reshape(1, 3 * H)
    bhh0 = gru_bhh0.reshape(1, 3 * H)
    bih1 = gru_bih1.reshape(1, 3 * H)
    bhh1 = gru_bhh1.reshape(1, 3 * H)
    linWT = jnp.zeros((H, H), jnp.float32).at[:, :2].set(lin_W.T)
    linb = jnp.zeros((1, H), jnp.float32).at[:, :2].set(lin_b.reshape(1, 2))

    cnt = _deg_kernel(dsti, z16, o16)
    scaled1, dinv = _prep_call(cnt[0], cnt[1], x_pad, conv_W0)
    agg1 = jnp.pad(_agg_kernel(scaled1, srci, dsti, z128), ((0, 0), (0, NPAD - NSC), (0, 0)))
    (scaled2,) = _mid_call(agg1[0], agg1[1], scaled1, dinv, b0, wih0T, bih0,
                           bhh0, conv_W1)
    agg2 = jnp.pad(_agg_kernel(scaled2, srci, dsti, z128), ((0, 0), (0, NPAD - NSC), (0, 0)))
    (res,) = _fin_call(agg2[0], agg2[1], scaled2, dinv, b1, wih1T, bih1,
                       bhh1, linWT, linb)
    return res[:N, :2]
